# baseline (speedup 1.0000x reference)
import jax
import jax.numpy as jnp
from jax import lax
from jax.experimental import pallas as pl
from jax.experimental.pallas import tpu as pltpu
from jax.experimental.pallas import tpu_sc as plsc

_N = 10000
_E = 160000
_D = 256
_DH = 512
_EPS = 1e-7

_NB = 4
_BLK = 128
_NS = 16
_CHUNK = 64
_GRP = 16
_NGRP = 10
_NBUF = 5
_CPT = _GRP * _NGRP
_EPT = _CPT * _CHUNK
_EPAD = _NS * _EPT
_TRASH = _N
_ACC_ROWS = 10240
_ZROWS = _ACC_ROWS // _NS
_FROWS = _ACC_ROWS // _NS
_R = 2000


def _gtab_body(x_ref, o_ref):
    f = jnp.maximum(x_ref[...], 0.0) + _EPS
    e = jnp.exp(f)
    fe = f * e
    o_ref[0] = e[:, :_BLK]
    o_ref[1] = e[:, _BLK:]
    o_ref[2] = fe[:, :_BLK]
    o_ref[3] = fe[:, _BLK:]


def _gtab(x):
    return pl.pallas_call(
        _gtab_body,
        grid=(_N // _R,),
        in_specs=[pl.BlockSpec((_R, _D), lambda i: (i, 0))],
        out_specs=pl.BlockSpec((_NB, _R, _BLK), lambda i: (0, i, 0)),
        out_shape=jax.ShapeDtypeStruct((_NB, _N, _BLK), jnp.float32),
    )(x)


def _seg_body(g_hbm, src_hbm, dst_hbm, zeros_hbm, out_hbm,
              srcv, dstv, rows_a, rows_b, rows_c, rows_d, rows_e, acc_sh,
              sem_a, sem_b, sem_c, sem_d, sem_e):
    c = lax.axis_index("c")
    s = lax.axis_index("s")
    rows = (rows_a, rows_b, rows_c, rows_d, rows_e)
    sems = (sem_a, sem_b, sem_c, sem_d, sem_e)
    for i in range(2):
        b = c * 2 + i
        pltpu.sync_copy(zeros_hbm, acc_sh.at[pl.ds(s * _ZROWS, _ZROWS)])
        plsc.subcore_barrier()

        def _group(g, carry):
            pltpu.sync_copy(src_hbm.at[b, s, g], srcv)
            pltpu.sync_copy(dst_hbm.at[s, g], dstv)
            for j in range(_NBUF):
                pltpu.async_copy(g_hbm.at[srcv.at[j]], rows[j], sems[j])
            for j in range(_GRP):
                k = j % _NBUF
                pltpu.make_async_copy(
                    g_hbm.at[srcv.at[0]], rows[k], sems[k]).wait()
                pltpu.sync_copy(rows[k], acc_sh.at[dstv.at[j]], add=True)
                if j + _NBUF < _GRP:
                    pltpu.async_copy(
                        g_hbm.at[srcv.at[j + _NBUF]], rows[k], sems[k])
            return carry

        lax.fori_loop(0, _NGRP, _group, 0)
        plsc.subcore_barrier()
        pltpu.sync_copy(acc_sh.at[pl.ds(s * _FROWS, _FROWS)],
                        out_hbm.at[b, pl.ds(s * _FROWS, _FROWS)])
        plsc.subcore_barrier()


def _seg_sum(g2d, src_all, dst_t, zeros):
    mesh = plsc.VectorSubcoreMesh(core_axis_name="c", subcore_axis_name="s")
    f = pl.kernel(
        _seg_body,
        out_type=jax.ShapeDtypeStruct((_NB, _ACC_ROWS, _BLK), jnp.float32),
        mesh=mesh,
        scratch_types=[
            pltpu.VMEM((_GRP, _CHUNK), jnp.int32),
            pltpu.VMEM((_GRP, _CHUNK), jnp.int32),
            pltpu.VMEM((_CHUNK, _BLK), jnp.float32),
            pltpu.VMEM((_CHUNK, _BLK), jnp.float32),
            pltpu.VMEM((_CHUNK, _BLK), jnp.float32),
            pltpu.VMEM((_CHUNK, _BLK), jnp.float32),
            pltpu.VMEM((_CHUNK, _BLK), jnp.float32),
            pltpu.VMEM_SHARED((_ACC_ROWS, _BLK), jnp.float32),
            pltpu.SemaphoreType.DMA,
            pltpu.SemaphoreType.DMA,
            pltpu.SemaphoreType.DMA,
            pltpu.SemaphoreType.DMA,
            pltpu.SemaphoreType.DMA,
        ],
    )
    return f(g2d, src_all, dst_t, zeros)


def _mlp1_body(s_ref, x_ref, w1_ref, b1_ref, h1_ref, st_ref):
    sb = s_ref[...]
    s1 = jnp.concatenate([sb[0], sb[1]], axis=1)
    s2 = jnp.concatenate([sb[2], sb[3]], axis=1)
    aggr = s2 / (s1 + 1e-16)
    h = aggr + x_ref[...]
    h1 = jnp.dot(h, w1_ref[...], preferred_element_type=jnp.float32) + b1_ref[...]
    h1_ref[...] = h1

    @pl.when(pl.program_id(0) == 0)
    def _():
        st_ref[...] = jnp.zeros_like(st_ref)

    st_ref[0:1] += jnp.sum(h1, axis=0, keepdims=True)
    st_ref[1:2] += jnp.sum(h1 * h1, axis=0, keepdims=True)


def _mlp1(s, x, w1, b1):
    return pl.pallas_call(
        _mlp1_body,
        grid=(_N // _R,),
        in_specs=[
            pl.BlockSpec((_NB, _R, _BLK), lambda i: (0, i, 0)),
            pl.BlockSpec((_R, _D), lambda i: (i, 0)),
            pl.BlockSpec((_D, _DH), lambda i: (0, 0)),
            pl.BlockSpec((1, _DH), lambda i: (0, 0)),
        ],
        out_specs=[
            pl.BlockSpec((_R, _DH), lambda i: (i, 0)),
            pl.BlockSpec((8, _DH), lambda i: (0, 0)),
        ],
        out_shape=[
            jax.ShapeDtypeStruct((_N, _DH), jnp.float32),
            jax.ShapeDtypeStruct((8, _DH), jnp.float32),
        ],
    )(s, x, w1, b1)


def _mlp2_body(h1_ref, st_ref, gam_ref, bet_ref, w2_ref, b2_ref, x_ref, o_ref):
    st = st_ref[...]
    mean = st[0:1] / _N
    var = st[1:2] / _N - mean * mean
    rstd = lax.rsqrt(var + 1e-5)
    t = (h1_ref[...] - mean) * (rstd * gam_ref[...]) + bet_ref[...]
    t = jnp.maximum(t, 0.0)
    y = jnp.dot(t, w2_ref[...], preferred_element_type=jnp.float32) + b2_ref[...]
    o_ref[...] = x_ref[...] + jnp.maximum(y, 0.0)


def _mlp2(h1, st, gamma, beta, w2, b2, x):
    return pl.pallas_call(
        _mlp2_body,
        grid=(_N // _R,),
        in_specs=[
            pl.BlockSpec((_R, _DH), lambda i: (i, 0)),
            pl.BlockSpec((8, _DH), lambda i: (0, 0)),
            pl.BlockSpec((1, _DH), lambda i: (0, 0)),
            pl.BlockSpec((1, _DH), lambda i: (0, 0)),
            pl.BlockSpec((_DH, _D), lambda i: (0, 0)),
            pl.BlockSpec((1, _D), lambda i: (0, 0)),
            pl.BlockSpec((_R, _D), lambda i: (i, 0)),
        ],
        out_specs=pl.BlockSpec((_R, _D), lambda i: (i, 0)),
        out_shape=jax.ShapeDtypeStruct((_N, _D), jnp.float32),
    )(h1, st, gamma, beta, w2, b2, x)


def kernel(x, edge_index, W1, b1, gamma, beta, W2, b2):
    src = edge_index[0].astype(jnp.int32)
    dst = edge_index[1].astype(jnp.int32)

    npad = _EPAD - _E
    pad_src = jnp.arange(npad, dtype=jnp.int32) % _N
    src_p = jnp.concatenate([src, pad_src])
    pad_dst = _TRASH + jnp.arange(npad, dtype=jnp.int32) % (_ACC_ROWS - _N)
    dst_p = jnp.concatenate([dst, pad_dst])
    src_t = src_p.reshape(_NS, _NGRP, _GRP, _CHUNK)
    dst_t = dst_p.reshape(_NS, _NGRP, _GRP, _CHUNK)
    src_all = src_t[None] + (jnp.arange(_NB, dtype=jnp.int32) * _N)[
        :, None, None, None, None]
    zeros = jnp.zeros((_ZROWS, _BLK), jnp.float32)

    g = _gtab(x)
    g2d = g.reshape(_NB * _N, _BLK)
    s = _seg_sum(g2d, src_all, dst_t, zeros)
    h1, st = _mlp1(s, x, W1, b1.reshape(1, _DH))
    return _mlp2(h1, st, gamma.reshape(1, _DH), beta.reshape(1, _DH),
                 W2, b2.reshape(1, _D), x)


# speedup vs baseline: 1.0145x; 1.0145x over previous
import jax
import jax.numpy as jnp
from jax import lax
from jax.experimental import pallas as pl
from jax.experimental.pallas import tpu as pltpu
from jax.experimental.pallas import tpu_sc as plsc

_N = 10000
_E = 160000
_D = 256
_DH = 512
_EPS = 1e-7

_NB = 4
_BLK = 128
_NS = 16
_CHUNK = 64
_GRP = 16
_NGRP = 10
_NBUF = 4
_CPT = _GRP * _NGRP
_EPT = _CPT * _CHUNK
_EPAD = _NS * _EPT
_TRASH = _N
_ACC_ROWS = 10240
_ZROWS = _ACC_ROWS // _NS
_FROWS = _ACC_ROWS // _NS
_R = 2000


def _gtab_body(x_ref, o_ref):
    f = jnp.maximum(x_ref[...], 0.0) + _EPS
    e = jnp.exp(f)
    fe = f * e
    o_ref[0] = e[:, :_BLK]
    o_ref[1] = e[:, _BLK:]
    o_ref[2] = fe[:, :_BLK]
    o_ref[3] = fe[:, _BLK:]


def _gtab(x):
    return pl.pallas_call(
        _gtab_body,
        grid=(_N // _R,),
        in_specs=[pl.BlockSpec((_R, _D), lambda i: (i, 0))],
        out_specs=pl.BlockSpec((_NB, _R, _BLK), lambda i: (0, i, 0)),
        out_shape=jax.ShapeDtypeStruct((_NB, _N, _BLK), jnp.float32),
    )(x)


def _seg_body(g_hbm, src_hbm, dst_hbm, zeros_hbm, out_hbm,
              srcv, dstv, rows_a, rows_b, rows_c, rows_d, acc_sh,
              sem_a, sem_b, sem_c, sem_d):
    c = lax.axis_index("c")
    s = lax.axis_index("s")
    rows = (rows_a, rows_b, rows_c, rows_d)
    sems = (sem_a, sem_b, sem_c, sem_d)
    for i in range(2):
        b = c * 2 + i
        pltpu.sync_copy(zeros_hbm, acc_sh.at[pl.ds(s * _ZROWS, _ZROWS)])
        plsc.subcore_barrier()

        def _group(g, carry):
            pltpu.sync_copy(src_hbm.at[b, s, g], srcv)
            pltpu.sync_copy(dst_hbm.at[s, g], dstv)
            for j in range(_NBUF):
                pltpu.async_copy(g_hbm.at[srcv.at[j]], rows[j], sems[j])
            for j in range(_GRP):
                k = j % _NBUF
                pltpu.make_async_copy(
                    g_hbm.at[srcv.at[0]], rows[k], sems[k]).wait()
                pltpu.sync_copy(rows[k], acc_sh.at[dstv.at[j]], add=True)
                if j + _NBUF < _GRP:
                    pltpu.async_copy(
                        g_hbm.at[srcv.at[j + _NBUF]], rows[k], sems[k])
            return carry

        lax.fori_loop(0, _NGRP, _group, 0)
        plsc.subcore_barrier()
        pltpu.sync_copy(acc_sh.at[pl.ds(s * _FROWS, _FROWS)],
                        out_hbm.at[b, pl.ds(s * _FROWS, _FROWS)])
        plsc.subcore_barrier()


def _seg_sum(g2d, src_all, dst_t, zeros):
    mesh = plsc.VectorSubcoreMesh(core_axis_name="c", subcore_axis_name="s")
    f = pl.kernel(
        _seg_body,
        out_type=jax.ShapeDtypeStruct((_NB, _ACC_ROWS, _BLK), jnp.float32),
        mesh=mesh,
        scratch_types=[
            pltpu.VMEM((_GRP, _CHUNK), jnp.int32),
            pltpu.VMEM((_GRP, _CHUNK), jnp.int32),
            pltpu.VMEM((_CHUNK, _BLK), jnp.float32),
            pltpu.VMEM((_CHUNK, _BLK), jnp.float32),
            pltpu.VMEM((_CHUNK, _BLK), jnp.float32),
            pltpu.VMEM((_CHUNK, _BLK), jnp.float32),
            pltpu.VMEM_SHARED((_ACC_ROWS, _BLK), jnp.float32),
            pltpu.SemaphoreType.DMA,
            pltpu.SemaphoreType.DMA,
            pltpu.SemaphoreType.DMA,
            pltpu.SemaphoreType.DMA,
        ],
    )
    return f(g2d, src_all, dst_t, zeros)


def _mlp1_body(s_ref, x_ref, w1_ref, b1_ref, h1_ref, st_ref):
    sb = s_ref[...]
    s1 = jnp.concatenate([sb[0], sb[1]], axis=1)
    s2 = jnp.concatenate([sb[2], sb[3]], axis=1)
    aggr = s2 / (s1 + 1e-16)
    h = aggr + x_ref[...]
    h1 = jnp.dot(h, w1_ref[...], preferred_element_type=jnp.float32) + b1_ref[...]
    h1_ref[...] = h1

    @pl.when(pl.program_id(0) == 0)
    def _():
        st_ref[...] = jnp.zeros_like(st_ref)

    st_ref[0:1] += jnp.sum(h1, axis=0, keepdims=True)
    st_ref[1:2] += jnp.sum(h1 * h1, axis=0, keepdims=True)


def _mlp1(s, x, w1, b1):
    return pl.pallas_call(
        _mlp1_body,
        grid=(_N // _R,),
        in_specs=[
            pl.BlockSpec((_NB, _R, _BLK), lambda i: (0, i, 0)),
            pl.BlockSpec((_R, _D), lambda i: (i, 0)),
            pl.BlockSpec((_D, _DH), lambda i: (0, 0)),
            pl.BlockSpec((1, _DH), lambda i: (0, 0)),
        ],
        out_specs=[
            pl.BlockSpec((_R, _DH), lambda i: (i, 0)),
            pl.BlockSpec((8, _DH), lambda i: (0, 0)),
        ],
        out_shape=[
            jax.ShapeDtypeStruct((_N, _DH), jnp.float32),
            jax.ShapeDtypeStruct((8, _DH), jnp.float32),
        ],
    )(s, x, w1, b1)


def _mlp2_body(h1_ref, st_ref, gam_ref, bet_ref, w2_ref, b2_ref, x_ref, o_ref):
    st = st_ref[...]
    mean = st[0:1] / _N
    var = st[1:2] / _N - mean * mean
    rstd = lax.rsqrt(var + 1e-5)
    t = (h1_ref[...] - mean) * (rstd * gam_ref[...]) + bet_ref[...]
    t = jnp.maximum(t, 0.0)
    y = jnp.dot(t, w2_ref[...], preferred_element_type=jnp.float32) + b2_ref[...]
    o_ref[...] = x_ref[...] + jnp.maximum(y, 0.0)


def _mlp2(h1, st, gamma, beta, w2, b2, x):
    return pl.pallas_call(
        _mlp2_body,
        grid=(_N // _R,),
        in_specs=[
            pl.BlockSpec((_R, _DH), lambda i: (i, 0)),
            pl.BlockSpec((8, _DH), lambda i: (0, 0)),
            pl.BlockSpec((1, _DH), lambda i: (0, 0)),
            pl.BlockSpec((1, _DH), lambda i: (0, 0)),
            pl.BlockSpec((_DH, _D), lambda i: (0, 0)),
            pl.BlockSpec((1, _D), lambda i: (0, 0)),
            pl.BlockSpec((_R, _D), lambda i: (i, 0)),
        ],
        out_specs=pl.BlockSpec((_R, _D), lambda i: (i, 0)),
        out_shape=jax.ShapeDtypeStruct((_N, _D), jnp.float32),
    )(h1, st, gamma, beta, w2, b2, x)


def kernel(x, edge_index, W1, b1, gamma, beta, W2, b2):
    src = edge_index[0].astype(jnp.int32)
    dst = edge_index[1].astype(jnp.int32)

    npad = _EPAD - _E
    pad_src = jnp.arange(npad, dtype=jnp.int32) % _N
    src_p = jnp.concatenate([src, pad_src])
    pad_dst = _TRASH + jnp.arange(npad, dtype=jnp.int32) % (_ACC_ROWS - _N)
    dst_p = jnp.concatenate([dst, pad_dst])
    src_t = src_p.reshape(_NS, _NGRP, _GRP, _CHUNK)
    dst_t = dst_p.reshape(_NS, _NGRP, _GRP, _CHUNK)
    src_all = src_t[None] + (jnp.arange(_NB, dtype=jnp.int32) * _N)[
        :, None, None, None, None]
    zeros = jnp.zeros((_ZROWS, _BLK), jnp.float32)

    g = _gtab(x)
    g2d = g.reshape(_NB * _N, _BLK)
    s = _seg_sum(g2d, src_all, dst_t, zeros)
    h1, st = _mlp1(s, x, W1, b1.reshape(1, _DH))
    return _mlp2(h1, st, gamma.reshape(1, _DH), beta.reshape(1, _DH),
                 W2, b2.reshape(1, _D), x)
